# baseline (speedup 1.0000x reference)
import jax
import jax.numpy as jnp
from jax.experimental import pallas as pl
from jax.experimental.pallas import tpu as pltpu

LABEL_SIZE = 1000
HIDDEN = 768
EPS = 1e-12


def _bcast_body(w_ref, gamma_ref, beta_ref, out_ref, ln_ref):
    i = pl.program_id(0)

    @pl.when(i == 0)
    def _():
        x = w_ref[...]
        mu = jnp.mean(x, axis=-1, keepdims=True)
        var = jnp.mean(jnp.square(x - mu), axis=-1, keepdims=True)
        ln_ref[...] = (x - mu) * jax.lax.rsqrt(var + EPS) * gamma_ref[...] + beta_ref[...]

    out_ref[...] = ln_ref[...][None, :, :]


def kernel(input_ids, W, gamma, beta):
    batch = input_ids.shape[0]
    out = pl.pallas_call(
        _bcast_body,
        grid=(batch,),
        in_specs=[
            pl.BlockSpec((LABEL_SIZE, HIDDEN), lambda i: (0, 0)),
            pl.BlockSpec((HIDDEN,), lambda i: (0,)),
            pl.BlockSpec((HIDDEN,), lambda i: (0,)),
        ],
        out_specs=pl.BlockSpec((1, LABEL_SIZE, HIDDEN), lambda i: (i, 0, 0)),
        out_shape=jax.ShapeDtypeStruct((batch, LABEL_SIZE, HIDDEN), jnp.float32),
        scratch_shapes=[pltpu.VMEM((LABEL_SIZE, HIDDEN), jnp.float32)],
    )(W, gamma, beta)
    return out
